# baseline (speedup 1.0000x reference)
import jax
import jax.numpy as jnp
from jax import lax
from jax.experimental import pallas as pl
from jax.experimental.pallas import tpu as pltpu
from jax.experimental.pallas import tpu_sc as plsc

N = 10000
E = 160000
D = 128
D2 = 2 * D
D3 = 3 * D
NC = 2
NS = 16
L = 16
EPT = E // NS
CH = 128
NFULL = EPT // CH
TAIL = EPT - NFULL * CH
DEGW = 128
EPS = 1e-5
NZCH = N // CH
NZTAIL = N - NZCH * CH

_mesh = lambda: plsc.VectorSubcoreMesh(core_axis_name="c", subcore_axis_name="s")


def _zero_rows(buf, rows, width):
    def fill(r, carry):
        for k in range(width // L):
            buf[r, pl.ds(k * L, L)] = jnp.zeros((L,), jnp.float32)
        return carry

    lax.fori_loop(0, rows, fill, 0)


def _zero_shared(s, zbuf, shared, rows):
    nch = N // rows
    tail = N - nch * rows
    for t in range(nch // NS + 1):
        z = s + NS * t

        @pl.when(z < nch)
        def _full():
            pltpu.sync_copy(zbuf, shared.at[pl.ds(z * rows, rows)])

        if tail:
            @pl.when(z == nch)
            def _tail():
                pltpu.sync_copy(zbuf.at[pl.ds(0, tail)],
                                shared.at[pl.ds(nch * rows, tail)])


def _emb_body(emb, nodes, x_out, idx_v, idx_t, rows_v, sem):
    c = lax.axis_index("c")
    s = lax.axis_index("s")
    wid = s * NC + c
    for t in range(3):
        z = wid + 32 * t

        @pl.when(z < NZCH)
        def _full():
            off = pl.multiple_of(z * CH, 8)
            pltpu.sync_copy(nodes.at[pl.ds(off, CH)], idx_v)
            pltpu.async_copy(emb.at[idx_v], rows_v, sem).wait()
            pltpu.sync_copy(rows_v, x_out.at[pl.ds(off, CH)])

        @pl.when(z == NZCH)
        def _tail():
            off = pl.multiple_of(NZCH * CH, 8)
            pltpu.sync_copy(nodes.at[pl.ds(off, NZTAIL)], idx_t)
            pltpu.async_copy(emb.at[idx_t], rows_v.at[pl.ds(0, NZTAIL)],
                             sem).wait()
            pltpu.sync_copy(rows_v.at[pl.ds(0, NZTAIL)],
                            x_out.at[pl.ds(off, NZTAIL)])


def _emb_gather(emb, nodes):
    return pl.kernel(
        _emb_body,
        out_type=jax.ShapeDtypeStruct((N, D), jnp.float32),
        mesh=_mesh(),
        scratch_types=[
            pltpu.VMEM((CH,), jnp.int32),
            pltpu.VMEM((NZTAIL,), jnp.int32),
            pltpu.VMEM((CH, D), jnp.float32),
            pltpu.SemaphoreType.DMA,
        ],
    )(emb, nodes)


def _deg_body(e0, e1, deg_out, idx2, idxt, ones_v, shared, sem):
    c = lax.axis_index("c")
    s = lax.axis_index("s")

    _zero_rows(ones_v, CH, DEGW)
    _zero_shared(s, ones_v, shared, CH)

    def fill1(r, carry):
        for k in range(DEGW // L):
            ones_v[r, pl.ds(k * L, L)] = jnp.ones((L,), jnp.float32)
        return carry

    lax.fori_loop(0, CH, fill1, 0)
    plsc.subcore_barrier()

    base = s * EPT

    def path(seg, a):
        def chunk(ck, carry):
            off = pl.multiple_of(base + ck * CH, 8)
            pltpu.sync_copy(seg.at[pl.ds(off, CH)], idx2.at[0])
            pltpu.sync_copy(ones_v, shared.at[idx2.at[0]], add=True)
            return carry

        lax.fori_loop(0, NFULL, chunk, 0)
        off = pl.multiple_of(base + NFULL * CH, 8)
        pltpu.sync_copy(seg.at[pl.ds(off, TAIL)], idxt.at[0])
        pltpu.sync_copy(ones_v.at[pl.ds(0, TAIL)], shared.at[idxt.at[0]],
                        add=True)
        plsc.subcore_barrier()

        @pl.when(s == 0)
        def _done():
            pltpu.sync_copy(shared, deg_out.at[a])

    @pl.when(c == 0)
    def _parent():
        path(e1, 0)

    @pl.when(c == 1)
    def _child():
        path(e0, 1)


def _deg_tables(e0, e1):
    return pl.kernel(
        _deg_body,
        out_type=jax.ShapeDtypeStruct((NC, N, DEGW), jnp.float32),
        mesh=_mesh(),
        scratch_types=[
            pltpu.VMEM((1, CH), jnp.int32),
            pltpu.VMEM((1, TAIL), jnp.int32),
            pltpu.VMEM((CH, DEGW), jnp.float32),
            pltpu.VMEM_SHARED((N, DEGW), jnp.float32),
            pltpu.SemaphoreType.DMA,
        ],
    )(e0, e1)


def _gather_body(ti, tj, e0, e1, s_out, stats, eit, ejt,
                 bufa0, bufb0, bufs0, bufa1, bufb1, bufs1, stat_v,
                 sga0, sgb0, sga1, sgb1, sw0, sw1):
    c = lax.axis_index("c")
    s = lax.axis_index("s")
    base = s * EPT

    zero = jnp.zeros((L,), jnp.float32)
    acc0 = tuple(zero for _ in range(2 * (D // L)))

    def path(ei, ej, a):
        tia = ti.at[a]
        tja = tj.at[a]
        pltpu.sync_copy(ei.at[pl.ds(base, EPT)], eit)
        pltpu.sync_copy(ej.at[pl.ds(base, EPT)], ejt)

        def issue(ck, ba, bb, sa, sb):
            o = ck * CH
            pltpu.async_copy(tia.at[eit.at[pl.ds(o, CH)]], ba, sa)
            pltpu.async_copy(tja.at[ejt.at[pl.ds(o, CH)]], bb, sb)

        def wait_gather(ba, bb, sa, sb):
            pltpu.make_async_copy(tia.at[eit.at[pl.ds(0, CH)]], ba, sa).wait()
            pltpu.make_async_copy(tja.at[ejt.at[pl.ds(0, CH)]], bb, sb).wait()

        def compute(n, ba, bb, bs, acc):
            def row(r, acc_):
                acc_ = list(acc_)
                for k in range(D // L):
                    sl = pl.ds(k * L, L)
                    sv = ba[r, sl] + bb[r, sl]
                    bs[r, sl] = sv
                    acc_[k] = acc_[k] + sv
                    acc_[(D // L) + k] = acc_[(D // L) + k] + sv * sv
                return tuple(acc_)

            return lax.fori_loop(0, n, row, acc)

        def wait_write(bs, sw):
            pltpu.make_async_copy(bs, s_out.at[a, pl.ds(0, CH)], sw).wait()

        issue(0, bufa0, bufb0, sga0, sgb0)

        def pair(p, acc):
            c0 = 2 * p
            c1 = 2 * p + 1
            issue(c1, bufa1, bufb1, sga1, sgb1)
            wait_gather(bufa0, bufb0, sga0, sgb0)

            @pl.when(p > 0)
            def _w0():
                wait_write(bufs0, sw0)

            acc = compute(CH, bufa0, bufb0, bufs0, acc)
            pltpu.async_copy(bufs0, s_out.at[a, pl.ds(base + c0 * CH, CH)],
                             sw0)

            @pl.when(p < NFULL // 2 - 1)
            def _nx():
                issue(c0 + 2, bufa0, bufb0, sga0, sgb0)

            wait_gather(bufa1, bufb1, sga1, sgb1)

            @pl.when(p > 0)
            def _w1():
                wait_write(bufs1, sw1)

            acc = compute(CH, bufa1, bufb1, bufs1, acc)
            pltpu.async_copy(bufs1, s_out.at[a, pl.ds(base + c1 * CH, CH)],
                             sw1)
            return acc

        acc = lax.fori_loop(0, NFULL // 2, pair, acc0)
        wait_write(bufs0, sw0)
        wait_write(bufs1, sw1)

        toff = NFULL * CH
        cpa = pltpu.async_copy(tia.at[eit.at[pl.ds(toff, TAIL)]],
                               bufa0.at[pl.ds(0, TAIL)], sga0)
        cpb = pltpu.async_copy(tja.at[ejt.at[pl.ds(toff, TAIL)]],
                               bufb0.at[pl.ds(0, TAIL)], sgb0)
        cpa.wait()
        cpb.wait()
        acc = compute(TAIL, bufa0, bufb0, bufs0, acc)
        pltpu.sync_copy(bufs0.at[pl.ds(0, TAIL)],
                        s_out.at[a, pl.ds(base + toff, TAIL)])

        for k in range(D // L):
            stat_v[0, pl.ds(k * L, L)] = acc[k]
            stat_v[1, pl.ds(k * L, L)] = acc[(D // L) + k]
        pltpu.sync_copy(stat_v, stats.at[a, s])

    @pl.when(c == 0)
    def _parent():
        path(e1, e0, 0)

    @pl.when(c == 1)
    def _child():
        path(e0, e1, 1)


def _edge_gather(ti, tj, e0, e1):
    return pl.kernel(
        _gather_body,
        out_type=[
            jax.ShapeDtypeStruct((NC, E, D), jnp.float32),
            jax.ShapeDtypeStruct((NC, NS, 2, D), jnp.float32),
        ],
        mesh=_mesh(),
        scratch_types=[
            pltpu.VMEM((EPT,), jnp.int32),
            pltpu.VMEM((EPT,), jnp.int32),
            pltpu.VMEM((CH, D), jnp.float32),
            pltpu.VMEM((CH, D), jnp.float32),
            pltpu.VMEM((CH, D), jnp.float32),
            pltpu.VMEM((CH, D), jnp.float32),
            pltpu.VMEM((CH, D), jnp.float32),
            pltpu.VMEM((CH, D), jnp.float32),
            pltpu.VMEM((2, D), jnp.float32),
            pltpu.SemaphoreType.DMA,
            pltpu.SemaphoreType.DMA,
            pltpu.SemaphoreType.DMA,
            pltpu.SemaphoreType.DMA,
            pltpu.SemaphoreType.DMA,
            pltpu.SemaphoreType.DMA,
        ],
    )(ti, tj, e0, e1)


CHS = 40
NCHS = E // CHS
NPAIRS = 125


def _scatter_body(h2, coef, e0, e1, out, idx_t, bufh0, bufh1, bufy0, bufy1,
                  coef_v, shared, sidx, sh0, sh1, ss0, ss1):
    c = lax.axis_index("c")
    s = lax.axis_index("s")

    _zero_rows(bufy0, CHS, D)
    _zero_shared(s, bufy0, shared, CHS)
    plsc.subcore_barrier()

    nt = 2 * NPAIRS
    himask = jnp.full((L,), -65536, jnp.int32)

    def path(seg, a):
        pltpu.sync_copy(coef.at[a], coef_v)
        k3 = [coef_v[0, pl.ds(k * L, L)] for k in range(D // L)]
        k4 = [coef_v[1, pl.ds(k * L, L)] for k in range(D // L)]

        def fire_idx(t, ring_row):
            o = pl.multiple_of((s + NS * t) * CHS, 8)
            pltpu.async_copy(seg.at[pl.ds(o, CHS)], idx_t.at[ring_row], sidx)

        def drain_idx():
            pltpu.make_async_copy(seg.at[pl.ds(0, CHS)], idx_t.at[0],
                                  sidx).wait()

        def issue_h2(t, bh, sh):
            z = s + NS * t
            pltpu.async_copy(h2.at[a, pl.ds(z * CHS, CHS)], bh, sh)

        def wait_h2(bh, sh):
            pltpu.make_async_copy(h2.at[a, pl.ds(0, CHS)], bh, sh).wait()

        def compute(n, bh, by):
            def row(r, carry):
                for k in range(D // 2 // L):
                    w = bh[r, pl.ds(k * L, L)]
                    lo = lax.bitcast_convert_type(w << 16, jnp.float32)
                    hi = lax.bitcast_convert_type(w & himask, jnp.float32)
                    ylo = jnp.maximum(lo * k3[2 * k] + k4[2 * k], 0.0)
                    yhi = jnp.maximum(hi * k3[2 * k + 1] + k4[2 * k + 1], 0.0)
                    by[r, pl.ds(2 * k * L, L)] = ylo
                    by[r, pl.ds((2 * k + 1) * L, L)] = yhi
                return carry

            lax.fori_loop(0, n, row, 0)

        def wait_scatter(by, ssem):
            pltpu.make_async_copy(by, shared.at[idx_t.at[0]], ssem).wait()

        fire_idx(0, 0)
        fire_idx(1, 1)
        issue_h2(0, bufh0, sh0)

        def pair(q, carry):
            t0 = 2 * q
            t1 = 2 * q + 1
            r0 = t0 & 3
            r1 = t1 & 3
            drain_idx()
            drain_idx()
            issue_h2(t1, bufh1, sh1)
            wait_h2(bufh0, sh0)

            @pl.when(q > 0)
            def _w0():
                wait_scatter(bufy0, ss0)

            compute(CHS, bufh0, bufy0)
            pltpu.async_copy(bufy0, shared.at[idx_t.at[r0]], ss0, add=True)

            @pl.when(q < NPAIRS - 1)
            def _nx():
                issue_h2(t0 + 2, bufh0, sh0)

            wait_h2(bufh1, sh1)

            @pl.when(q > 0)
            def _w1():
                wait_scatter(bufy1, ss1)

            @pl.when(q < NPAIRS - 1)
            def _pf():
                fire_idx(t0 + 2, (t0 + 2) & 3)
                fire_idx(t1 + 2, (t1 + 2) & 3)

            compute(CHS, bufh1, bufy1)
            pltpu.async_copy(bufy1, shared.at[idx_t.at[r1]], ss1, add=True)
            return carry

        lax.fori_loop(0, NPAIRS, pair, 0)
        wait_scatter(bufy0, ss0)
        wait_scatter(bufy1, ss1)
        plsc.subcore_barrier()

        @pl.when(s == 0)
        def _done():
            pltpu.sync_copy(shared, out.at[a])

    @pl.when(c == 0)
    def _parent():
        path(e1, 0)

    @pl.when(c == 1)
    def _child():
        path(e0, 1)


def _edge_scatter(h2v, coef, e0, e1):
    return pl.kernel(
        _scatter_body,
        out_type=jax.ShapeDtypeStruct((NC, N, D), jnp.float32),
        mesh=_mesh(),
        scratch_types=[
            pltpu.VMEM((4, CHS), jnp.int32),
            pltpu.VMEM((CHS, D // 2), jnp.int32),
            pltpu.VMEM((CHS, D // 2), jnp.int32),
            pltpu.VMEM((CHS, D), jnp.float32),
            pltpu.VMEM((CHS, D), jnp.float32),
            pltpu.VMEM((2, D), jnp.float32),
            pltpu.VMEM_SHARED((N, D), jnp.float32),
            pltpu.SemaphoreType.DMA,
            pltpu.SemaphoreType.DMA,
            pltpu.SemaphoreType.DMA,
            pltpu.SemaphoreType.DMA,
            pltpu.SemaphoreType.DMA,
        ],
    )(h2v, coef, e0, e1)


BN_NODE = 2000
BE = 2000


def _nodemm_body(x_ref, wi_ref, wj_ref, ti_ref, tj_ref):
    x = x_ref[...]
    dn = (((1,), (1,)), ((), ()))
    for a in range(NC):
        ti_ref[a] = lax.dot_general(x, wi_ref[a], dn,
                                    preferred_element_type=jnp.float32)
        tj_ref[a] = lax.dot_general(x, wj_ref[a], dn,
                                    preferred_element_type=jnp.float32)


def _node_tables(x, wi, wj):
    g = N // BN_NODE
    return pl.pallas_call(
        _nodemm_body,
        grid=(g,),
        in_specs=[
            pl.BlockSpec((BN_NODE, D), lambda i: (i, 0)),
            pl.BlockSpec((NC, D, D), lambda i: (0, 0, 0)),
            pl.BlockSpec((NC, D, D), lambda i: (0, 0, 0)),
        ],
        out_specs=[
            pl.BlockSpec((NC, BN_NODE, D), lambda i: (0, i, 0)),
            pl.BlockSpec((NC, BN_NODE, D), lambda i: (0, i, 0)),
        ],
        out_shape=[
            jax.ShapeDtypeStruct((NC, N, D), jnp.float32),
            jax.ShapeDtypeStruct((NC, N, D), jnp.float32),
        ],
    )(x, wi, wj)


def _mlp_body(s_ref, stats_ref, prm_ref, w2_ref, h2_ref, coef_ref, acc):
    e = pl.program_id(1)
    ne = pl.num_programs(1)
    st = jnp.sum(stats_ref[0], axis=0)
    mean = st[0:1] * (1.0 / E)
    var = st[1:2] * (1.0 / E) - mean * mean
    inv1 = lax.rsqrt(var + EPS)
    g1 = prm_ref[0, 0:1]
    be1 = prm_ref[0, 1:2]
    k1 = g1 * inv1
    k2 = be1 - mean * inv1 * g1
    h1 = jnp.maximum(s_ref[0] * k1 + k2, 0.0)
    h2 = lax.dot_general(h1, w2_ref[0], (((1,), (1,)), ((), ())),
                         preferred_element_type=jnp.float32)
    h2_ref[0] = h2.astype(jnp.bfloat16)

    @pl.when(e == 0)
    def _init():
        acc[...] = jnp.zeros_like(acc)

    acc[0:1, :] += jnp.sum(h2, axis=0, keepdims=True)
    acc[1:2, :] += jnp.sum(h2 * h2, axis=0, keepdims=True)

    @pl.when(e == ne - 1)
    def _fin():
        m2 = acc[0:1, :] * (1.0 / E)
        v2 = acc[1:2, :] * (1.0 / E) - m2 * m2
        inv2 = lax.rsqrt(v2 + EPS)
        g2 = prm_ref[0, 2:3]
        be2 = prm_ref[0, 3:4]
        coef_ref[0, 0:1] = g2 * inv2
        coef_ref[0, 1:2] = be2 - m2 * inv2 * g2


def _edge_mlp(s_edges, stats, prm, w2):
    ge = E // BE
    return pl.pallas_call(
        _mlp_body,
        grid=(NC, ge),
        in_specs=[
            pl.BlockSpec((1, BE, D), lambda a, e: (a, e, 0)),
            pl.BlockSpec((1, NS, 2, D), lambda a, e: (a, 0, 0, 0)),
            pl.BlockSpec((1, 4, D), lambda a, e: (a, 0, 0)),
            pl.BlockSpec((1, D, D), lambda a, e: (a, 0, 0)),
        ],
        out_specs=[
            pl.BlockSpec((1, BE, D), lambda a, e: (a, e, 0)),
            pl.BlockSpec((1, 2, D), lambda a, e: (a, 0, 0)),
        ],
        out_shape=[
            jax.ShapeDtypeStruct((NC, E, D), jnp.bfloat16),
            jax.ShapeDtypeStruct((NC, 2, D), jnp.float32),
        ],
        scratch_shapes=[pltpu.VMEM((2, D), jnp.float32)],
    )(s_edges, stats, prm, w2)


def _node_body(x_ref, agg_ref, deg_ref, wx_ref, wfi_ref, wfo_ref, fcb_ref,
               w2_ref, b2_ref, xn_ref):
    x = x_ref[...]
    dn = (((1,), (1,)), ((), ()))
    dp = deg_ref[0, :, 0:1]
    dc = deg_ref[1, :, 0:1]
    fi = agg_ref[0] * jnp.where(dp > 0, 1.0 / dp, 0.0)
    fo = agg_ref[1] * jnp.where(dc > 0, 1.0 / dc, 0.0)
    h = lax.dot_general(x, wx_ref[...], dn, preferred_element_type=jnp.float32)
    h += lax.dot_general(fi, wfi_ref[...], dn,
                         preferred_element_type=jnp.float32)
    h += lax.dot_general(fo, wfo_ref[...], dn,
                         preferred_element_type=jnp.float32)
    h = jnp.maximum(h + fcb_ref[...], 0.0)
    xn = lax.dot_general(h, w2_ref[...], dn, preferred_element_type=jnp.float32)
    xn_ref[...] = x + xn + b2_ref[...]


def _node_update(x, agg, deg, wx, wfi, wfo, fcb, w2, b2):
    g = N // BN_NODE
    return pl.pallas_call(
        _node_body,
        grid=(g,),
        in_specs=[
            pl.BlockSpec((BN_NODE, D), lambda i: (i, 0)),
            pl.BlockSpec((NC, BN_NODE, D), lambda i: (0, i, 0)),
            pl.BlockSpec((NC, BN_NODE, DEGW), lambda i: (0, i, 0)),
            pl.BlockSpec((D2, D), lambda i: (0, 0)),
            pl.BlockSpec((D2, D), lambda i: (0, 0)),
            pl.BlockSpec((D2, D), lambda i: (0, 0)),
            pl.BlockSpec((1, D2), lambda i: (0, 0)),
            pl.BlockSpec((D, D2), lambda i: (0, 0)),
            pl.BlockSpec((1, D), lambda i: (0, 0)),
        ],
        out_specs=pl.BlockSpec((BN_NODE, D), lambda i: (i, 0)),
        out_shape=jax.ShapeDtypeStruct((N, D), jnp.float32),
    )(x, agg, deg, wx, wfi, wfo, fcb, w2, b2)


def _final_body(x_ref, w_ref, b_ref, o_ref):
    o_ref[...] = lax.dot_general(
        x_ref[...], w_ref[...], (((1,), (1,)), ((), ())),
        preferred_element_type=jnp.float32) + b_ref[...]


def _final_conv(x, w, b):
    g = N // BN_NODE
    return pl.pallas_call(
        _final_body,
        grid=(g,),
        in_specs=[
            pl.BlockSpec((BN_NODE, D), lambda i: (i, 0)),
            pl.BlockSpec((D2, D), lambda i: (0, 0)),
            pl.BlockSpec((1, D2), lambda i: (0, 0)),
        ],
        out_specs=pl.BlockSpec((BN_NODE, D2), lambda i: (i, 0)),
        out_shape=jax.ShapeDtypeStruct((N, D2), jnp.float32),
    )(x, w, b)


@jax.jit
def _run(nodes, edges, emb, parent_w1, parent_b1, parent_g1, parent_be1,
         parent_w2, parent_b2, parent_g2, parent_be2, child_w1, child_b1,
         child_g1, child_be1, child_w2, child_b2, child_g2, child_be2,
         fc_w, fc_b, fc2_w, fc2_b, conv_w, conv_b):
    nodes = nodes.astype(jnp.int32)
    edges = edges.astype(jnp.int32)
    e0 = edges[0]
    e1 = edges[1]

    wi = jnp.stack([parent_w1[:, :D], child_w1[:, :D]])
    wj = jnp.stack([parent_w1[:, D:], child_w1[:, D:]])
    prm = jnp.stack([
        jnp.stack([parent_g1, parent_be1, parent_g2, parent_be2]),
        jnp.stack([child_g1, child_be1, child_g2, child_be2]),
    ])
    w2 = jnp.stack([parent_w2, child_w2])
    lane_perm = lambda w: w.reshape(D2, 4, 16, 2).transpose(0, 1, 3, 2) \
        .reshape(D2, D)
    wx = fc_w[:, :D]
    wfi = lane_perm(fc_w[:, D:D2])
    wfo = lane_perm(fc_w[:, D2:])
    fcb = fc_b.reshape(1, D2)
    fc2b = fc2_b.reshape(1, D)
    convb = conv_b.reshape(1, D2)

    x = _emb_gather(emb, nodes)
    deg = _deg_tables(e0, e1)

    for _ in range(2):
        ti, tj = _node_tables(x, wi, wj)
        s_edges, stats = _edge_gather(ti, tj, e0, e1)
        h2, coef = _edge_mlp(s_edges, stats, prm, w2)
        h2v = lax.bitcast_convert_type(h2.reshape(NC, E, D // 2, 2),
                                       jnp.int32)
        coefp = coef.reshape(NC, 2, 4, 16, 2).transpose(0, 1, 2, 4, 3) \
            .reshape(NC, 2, D)
        agg = _edge_scatter(h2v, coefp, e0, e1)
        x = _node_update(x, agg, deg, wx, wfi, wfo, fcb, fc2_w, fc2b)

    return _final_conv(x, conv_w, convb)


def kernel(nodes, edges, emb, parent_w1, parent_b1, parent_g1, parent_be1,
           parent_w2, parent_b2, parent_g2, parent_be2, child_w1, child_b1,
           child_g1, child_be1, child_w2, child_b2, child_g2, child_be2,
           fc_w, fc_b, fc2_w, fc2_b, conv_w, conv_b):
    return _run(nodes, edges, emb, parent_w1, parent_b1, parent_g1, parent_be1,
                parent_w2, parent_b2, parent_g2, parent_be2, child_w1,
                child_b1, child_g1, child_be1, child_w2, child_b2, child_g2,
                child_be2, fc_w, fc_b, fc2_w, fc2_b, conv_w, conv_b)


# speedup vs baseline: 1.7952x; 1.7952x over previous
import jax
import jax.numpy as jnp
from jax import lax
from jax.experimental import pallas as pl
from jax.experimental.pallas import tpu as pltpu
from jax.experimental.pallas import tpu_sc as plsc

N = 10000
E = 160000
D = 128
D2 = 2 * D
D3 = 3 * D
NC = 2
NS = 16
L = 16
EPT = E // NS
CH = 128
NFULL = EPT // CH
TAIL = EPT - NFULL * CH
DEGW = 128
EPS = 1e-5
NZCH = N // CH
NZTAIL = N - NZCH * CH

_mesh = lambda: plsc.VectorSubcoreMesh(core_axis_name="c", subcore_axis_name="s")


def _zero_rows(buf, rows, width):
    def fill(r, carry):
        for k in range(width // L):
            buf[r, pl.ds(k * L, L)] = jnp.zeros((L,), jnp.float32)
        return carry

    lax.fori_loop(0, rows, fill, 0)


def _zero_shared(s, zbuf, shared, rows):
    nch = N // rows
    tail = N - nch * rows
    for t in range(nch // NS + 1):
        z = s + NS * t

        @pl.when(z < nch)
        def _full():
            pltpu.sync_copy(zbuf, shared.at[pl.ds(z * rows, rows)])

        if tail:
            @pl.when(z == nch)
            def _tail():
                pltpu.sync_copy(zbuf.at[pl.ds(0, tail)],
                                shared.at[pl.ds(nch * rows, tail)])


def _emb_body(emb, nodes, x_out, idx_v, idx_t, rows_v, sem):
    c = lax.axis_index("c")
    s = lax.axis_index("s")
    wid = s * NC + c
    for t in range(3):
        z = wid + 32 * t

        @pl.when(z < NZCH)
        def _full():
            off = pl.multiple_of(z * CH, 8)
            pltpu.sync_copy(nodes.at[pl.ds(off, CH)], idx_v)
            pltpu.async_copy(emb.at[idx_v], rows_v, sem).wait()
            pltpu.sync_copy(rows_v, x_out.at[pl.ds(off, CH)])

        @pl.when(z == NZCH)
        def _tail():
            off = pl.multiple_of(NZCH * CH, 8)
            pltpu.sync_copy(nodes.at[pl.ds(off, NZTAIL)], idx_t)
            pltpu.async_copy(emb.at[idx_t], rows_v.at[pl.ds(0, NZTAIL)],
                             sem).wait()
            pltpu.sync_copy(rows_v.at[pl.ds(0, NZTAIL)],
                            x_out.at[pl.ds(off, NZTAIL)])


def _emb_gather(emb, nodes):
    return pl.kernel(
        _emb_body,
        out_type=jax.ShapeDtypeStruct((N, D), jnp.float32),
        mesh=_mesh(),
        scratch_types=[
            pltpu.VMEM((CH,), jnp.int32),
            pltpu.VMEM((NZTAIL,), jnp.int32),
            pltpu.VMEM((CH, D), jnp.float32),
            pltpu.SemaphoreType.DMA,
        ],
    )(emb, nodes)


def _deg_body(e0, e1, deg_out, idx2, idxt, ones_v, shared, sem):
    c = lax.axis_index("c")
    s = lax.axis_index("s")

    _zero_rows(ones_v, CH, DEGW)
    _zero_shared(s, ones_v, shared, CH)

    def fill1(r, carry):
        for k in range(DEGW // L):
            ones_v[r, pl.ds(k * L, L)] = jnp.ones((L,), jnp.float32)
        return carry

    lax.fori_loop(0, CH, fill1, 0)
    plsc.subcore_barrier()

    base = s * EPT

    def path(seg, a):
        def chunk(ck, carry):
            off = pl.multiple_of(base + ck * CH, 8)
            pltpu.sync_copy(seg.at[pl.ds(off, CH)], idx2.at[0])
            pltpu.sync_copy(ones_v, shared.at[idx2.at[0]], add=True)
            return carry

        lax.fori_loop(0, NFULL, chunk, 0)
        off = pl.multiple_of(base + NFULL * CH, 8)
        pltpu.sync_copy(seg.at[pl.ds(off, TAIL)], idxt.at[0])
        pltpu.sync_copy(ones_v.at[pl.ds(0, TAIL)], shared.at[idxt.at[0]],
                        add=True)
        plsc.subcore_barrier()

        @pl.when(s == 0)
        def _done():
            pltpu.sync_copy(shared, deg_out.at[a])

    @pl.when(c == 0)
    def _parent():
        path(e1, 0)

    @pl.when(c == 1)
    def _child():
        path(e0, 1)


def _deg_tables(e0, e1):
    return pl.kernel(
        _deg_body,
        out_type=jax.ShapeDtypeStruct((NC, N, DEGW), jnp.float32),
        mesh=_mesh(),
        scratch_types=[
            pltpu.VMEM((1, CH), jnp.int32),
            pltpu.VMEM((1, TAIL), jnp.int32),
            pltpu.VMEM((CH, DEGW), jnp.float32),
            pltpu.VMEM_SHARED((N, DEGW), jnp.float32),
            pltpu.SemaphoreType.DMA,
        ],
    )(e0, e1)


def _gather_body(ti, tj, e0, e1, s_out, stats, eit, ejt,
                 bufa0, bufb0, bufs0, bufa1, bufb1, bufs1, stat_v,
                 sga0, sgb0, sga1, sgb1, sw0, sw1):
    c = lax.axis_index("c")
    s = lax.axis_index("s")
    base = s * EPT

    zero = jnp.zeros((L,), jnp.float32)
    acc0 = tuple(zero for _ in range(2 * (D // L)))

    def path(ei, ej, a):
        tia = ti.at[a]
        tja = tj.at[a]
        pltpu.sync_copy(ei.at[pl.ds(base, EPT)], eit)
        pltpu.sync_copy(ej.at[pl.ds(base, EPT)], ejt)

        def issue(ck, ba, bb, sa, sb):
            o = ck * CH
            pltpu.async_copy(tia.at[eit.at[pl.ds(o, CH)]], ba, sa)
            pltpu.async_copy(tja.at[ejt.at[pl.ds(o, CH)]], bb, sb)

        def wait_gather(ba, bb, sa, sb):
            pltpu.make_async_copy(tia.at[eit.at[pl.ds(0, CH)]], ba, sa).wait()
            pltpu.make_async_copy(tja.at[ejt.at[pl.ds(0, CH)]], bb, sb).wait()

        def compute(n, ba, bb, bs, acc):
            def row(r, acc_):
                acc_ = list(acc_)
                for k in range(D // L):
                    sl = pl.ds(k * L, L)
                    sv = ba[r, sl] + bb[r, sl]
                    bs[r, sl] = sv
                    acc_[k] = acc_[k] + sv
                    acc_[(D // L) + k] = acc_[(D // L) + k] + sv * sv
                return tuple(acc_)

            return lax.fori_loop(0, n, row, acc)

        def wait_write(bs, sw):
            pltpu.make_async_copy(bs, s_out.at[a, pl.ds(0, CH)], sw).wait()

        issue(0, bufa0, bufb0, sga0, sgb0)

        def pair(p, acc):
            c0 = 2 * p
            c1 = 2 * p + 1
            issue(c1, bufa1, bufb1, sga1, sgb1)
            wait_gather(bufa0, bufb0, sga0, sgb0)

            @pl.when(p > 0)
            def _w0():
                wait_write(bufs0, sw0)

            acc = compute(CH, bufa0, bufb0, bufs0, acc)
            pltpu.async_copy(bufs0, s_out.at[a, pl.ds(base + c0 * CH, CH)],
                             sw0)

            @pl.when(p < NFULL // 2 - 1)
            def _nx():
                issue(c0 + 2, bufa0, bufb0, sga0, sgb0)

            wait_gather(bufa1, bufb1, sga1, sgb1)

            @pl.when(p > 0)
            def _w1():
                wait_write(bufs1, sw1)

            acc = compute(CH, bufa1, bufb1, bufs1, acc)
            pltpu.async_copy(bufs1, s_out.at[a, pl.ds(base + c1 * CH, CH)],
                             sw1)
            return acc

        acc = lax.fori_loop(0, NFULL // 2, pair, acc0)
        wait_write(bufs0, sw0)
        wait_write(bufs1, sw1)

        toff = NFULL * CH
        cpa = pltpu.async_copy(tia.at[eit.at[pl.ds(toff, TAIL)]],
                               bufa0.at[pl.ds(0, TAIL)], sga0)
        cpb = pltpu.async_copy(tja.at[ejt.at[pl.ds(toff, TAIL)]],
                               bufb0.at[pl.ds(0, TAIL)], sgb0)
        cpa.wait()
        cpb.wait()
        acc = compute(TAIL, bufa0, bufb0, bufs0, acc)
        pltpu.sync_copy(bufs0.at[pl.ds(0, TAIL)],
                        s_out.at[a, pl.ds(base + toff, TAIL)])

        for k in range(D // L):
            stat_v[0, pl.ds(k * L, L)] = acc[k]
            stat_v[1, pl.ds(k * L, L)] = acc[(D // L) + k]
        pltpu.sync_copy(stat_v, stats.at[a, s])

    @pl.when(c == 0)
    def _parent():
        path(e1, e0, 0)

    @pl.when(c == 1)
    def _child():
        path(e0, e1, 1)


def _edge_gather(ti, tj, e0, e1):
    return pl.kernel(
        _gather_body,
        out_type=[
            jax.ShapeDtypeStruct((NC, E, D), jnp.float32),
            jax.ShapeDtypeStruct((NC, NS, 2, D), jnp.float32),
        ],
        mesh=_mesh(),
        scratch_types=[
            pltpu.VMEM((EPT,), jnp.int32),
            pltpu.VMEM((EPT,), jnp.int32),
            pltpu.VMEM((CH, D), jnp.float32),
            pltpu.VMEM((CH, D), jnp.float32),
            pltpu.VMEM((CH, D), jnp.float32),
            pltpu.VMEM((CH, D), jnp.float32),
            pltpu.VMEM((CH, D), jnp.float32),
            pltpu.VMEM((CH, D), jnp.float32),
            pltpu.VMEM((2, D), jnp.float32),
            pltpu.SemaphoreType.DMA,
            pltpu.SemaphoreType.DMA,
            pltpu.SemaphoreType.DMA,
            pltpu.SemaphoreType.DMA,
            pltpu.SemaphoreType.DMA,
            pltpu.SemaphoreType.DMA,
        ],
    )(ti, tj, e0, e1)


CHS = 40
NCHS = E // CHS
NPAIRS = 125


def _scatter_body(h2, coef, e0, e1, out, idx_t, bufh0, bufh1, bufy0, bufy1,
                  coef_v, shared, sidx, sh0, sh1, ss0, ss1):
    c = lax.axis_index("c")
    s = lax.axis_index("s")

    _zero_rows(bufy0, CHS, D)
    _zero_shared(s, bufy0, shared, CHS)
    plsc.subcore_barrier()

    nt = 2 * NPAIRS
    himask = jnp.full((L,), -65536, jnp.int32)

    def path(seg, a):
        pltpu.sync_copy(coef.at[a], coef_v)
        k3 = [coef_v[0, pl.ds(k * L, L)] for k in range(D // L)]
        k4 = [coef_v[1, pl.ds(k * L, L)] for k in range(D // L)]

        def fire_idx(t, ring_row):
            o = pl.multiple_of((s + NS * t) * CHS, 8)
            pltpu.async_copy(seg.at[pl.ds(o, CHS)], idx_t.at[ring_row], sidx)

        def drain_idx():
            pltpu.make_async_copy(seg.at[pl.ds(0, CHS)], idx_t.at[0],
                                  sidx).wait()

        def issue_h2(t, bh, sh):
            z = s + NS * t
            pltpu.async_copy(h2.at[a, pl.ds(z * CHS, CHS)], bh, sh)

        def wait_h2(bh, sh):
            pltpu.make_async_copy(h2.at[a, pl.ds(0, CHS)], bh, sh).wait()

        def compute(n, bh, by):
            def row(r, carry):
                for k in range(D // 2 // L):
                    w = bh[r, pl.ds(k * L, L)]
                    lo = lax.bitcast_convert_type(w << 16, jnp.float32)
                    hi = lax.bitcast_convert_type(w & himask, jnp.float32)
                    kk = (D // 2 // L) + k
                    ylo = jnp.maximum(lo * k3[k] + k4[k], 0.0)
                    yhi = jnp.maximum(hi * k3[kk] + k4[kk], 0.0)
                    by[r, pl.ds(k * L, L)] = ylo
                    by[r, pl.ds(kk * L, L)] = yhi
                return carry

            lax.fori_loop(0, n, row, 0)

        def wait_scatter(by, ssem):
            pltpu.make_async_copy(by, shared.at[idx_t.at[0]], ssem).wait()

        fire_idx(0, 0)
        fire_idx(1, 1)
        issue_h2(0, bufh0, sh0)

        def pair(q, carry):
            t0 = 2 * q
            t1 = 2 * q + 1
            r0 = t0 & 3
            r1 = t1 & 3
            drain_idx()
            drain_idx()
            issue_h2(t1, bufh1, sh1)
            wait_h2(bufh0, sh0)

            @pl.when(q > 0)
            def _w0():
                wait_scatter(bufy0, ss0)

            compute(CHS, bufh0, bufy0)
            pltpu.async_copy(bufy0, shared.at[idx_t.at[r0]], ss0, add=True)

            @pl.when(q < NPAIRS - 1)
            def _nx():
                issue_h2(t0 + 2, bufh0, sh0)

            wait_h2(bufh1, sh1)

            @pl.when(q > 0)
            def _w1():
                wait_scatter(bufy1, ss1)

            @pl.when(q < NPAIRS - 1)
            def _pf():
                fire_idx(t0 + 2, (t0 + 2) & 3)
                fire_idx(t1 + 2, (t1 + 2) & 3)

            compute(CHS, bufh1, bufy1)
            pltpu.async_copy(bufy1, shared.at[idx_t.at[r1]], ss1, add=True)
            return carry

        lax.fori_loop(0, NPAIRS, pair, 0)
        wait_scatter(bufy0, ss0)
        wait_scatter(bufy1, ss1)
        plsc.subcore_barrier()

        @pl.when(s == 0)
        def _done():
            pltpu.sync_copy(shared, out.at[a])

    @pl.when(c == 0)
    def _parent():
        path(e1, 0)

    @pl.when(c == 1)
    def _child():
        path(e0, 1)


def _edge_scatter(h2v, coef, e0, e1):
    return pl.kernel(
        _scatter_body,
        out_type=jax.ShapeDtypeStruct((NC, N, D), jnp.float32),
        mesh=_mesh(),
        scratch_types=[
            pltpu.VMEM((4, CHS), jnp.int32),
            pltpu.VMEM((CHS, D // 2), jnp.int32),
            pltpu.VMEM((CHS, D // 2), jnp.int32),
            pltpu.VMEM((CHS, D), jnp.float32),
            pltpu.VMEM((CHS, D), jnp.float32),
            pltpu.VMEM((2, D), jnp.float32),
            pltpu.VMEM_SHARED((N, D), jnp.float32),
            pltpu.SemaphoreType.DMA,
            pltpu.SemaphoreType.DMA,
            pltpu.SemaphoreType.DMA,
            pltpu.SemaphoreType.DMA,
            pltpu.SemaphoreType.DMA,
        ],
    )(h2v, coef, e0, e1)


BN_NODE = 2000
BE = 2000


def _nodemm_body(x_ref, wi_ref, wj_ref, ti_ref, tj_ref):
    x = x_ref[...]
    dn = (((1,), (1,)), ((), ()))
    for a in range(NC):
        ti_ref[a] = lax.dot_general(x, wi_ref[a], dn,
                                    preferred_element_type=jnp.float32)
        tj_ref[a] = lax.dot_general(x, wj_ref[a], dn,
                                    preferred_element_type=jnp.float32)


def _node_tables(x, wi, wj):
    g = N // BN_NODE
    return pl.pallas_call(
        _nodemm_body,
        grid=(g,),
        in_specs=[
            pl.BlockSpec((BN_NODE, D), lambda i: (i, 0)),
            pl.BlockSpec((NC, D, D), lambda i: (0, 0, 0)),
            pl.BlockSpec((NC, D, D), lambda i: (0, 0, 0)),
        ],
        out_specs=[
            pl.BlockSpec((NC, BN_NODE, D), lambda i: (0, i, 0)),
            pl.BlockSpec((NC, BN_NODE, D), lambda i: (0, i, 0)),
        ],
        out_shape=[
            jax.ShapeDtypeStruct((NC, N, D), jnp.float32),
            jax.ShapeDtypeStruct((NC, N, D), jnp.float32),
        ],
    )(x, wi, wj)


def _mlp_body(s_ref, stats_ref, prm_ref, w2_ref, h2_ref, coef_ref, acc):
    e = pl.program_id(1)
    ne = pl.num_programs(1)
    st = jnp.sum(stats_ref[0], axis=0)
    mean = st[0:1] * (1.0 / E)
    var = st[1:2] * (1.0 / E) - mean * mean
    inv1 = lax.rsqrt(var + EPS)
    g1 = prm_ref[0, 0:1]
    be1 = prm_ref[0, 1:2]
    k1 = g1 * inv1
    k2 = be1 - mean * inv1 * g1
    h1 = jnp.maximum(s_ref[0] * k1 + k2, 0.0)
    h2 = lax.dot_general(h1, w2_ref[0], (((1,), (1,)), ((), ())),
                         preferred_element_type=jnp.float32)
    bl = lax.bitcast_convert_type(h2[:, :D // 2], jnp.int32) + 0x8000
    bh = lax.bitcast_convert_type(h2[:, D // 2:], jnp.int32) + 0x8000
    h2_ref[0] = lax.shift_right_logical(bl, 16) | (bh & (-65536))

    @pl.when(e == 0)
    def _init():
        acc[...] = jnp.zeros_like(acc)

    acc[0:1, :] += jnp.sum(h2, axis=0, keepdims=True)
    acc[1:2, :] += jnp.sum(h2 * h2, axis=0, keepdims=True)

    @pl.when(e == ne - 1)
    def _fin():
        m2 = acc[0:1, :] * (1.0 / E)
        v2 = acc[1:2, :] * (1.0 / E) - m2 * m2
        inv2 = lax.rsqrt(v2 + EPS)
        g2 = prm_ref[0, 2:3]
        be2 = prm_ref[0, 3:4]
        coef_ref[0, 0:1] = g2 * inv2
        coef_ref[0, 1:2] = be2 - m2 * inv2 * g2


def _edge_mlp(s_edges, stats, prm, w2):
    ge = E // BE
    return pl.pallas_call(
        _mlp_body,
        grid=(NC, ge),
        in_specs=[
            pl.BlockSpec((1, BE, D), lambda a, e: (a, e, 0)),
            pl.BlockSpec((1, NS, 2, D), lambda a, e: (a, 0, 0, 0)),
            pl.BlockSpec((1, 4, D), lambda a, e: (a, 0, 0)),
            pl.BlockSpec((1, D, D), lambda a, e: (a, 0, 0)),
        ],
        out_specs=[
            pl.BlockSpec((1, BE, D // 2), lambda a, e: (a, e, 0)),
            pl.BlockSpec((1, 2, D), lambda a, e: (a, 0, 0)),
        ],
        out_shape=[
            jax.ShapeDtypeStruct((NC, E, D // 2), jnp.int32),
            jax.ShapeDtypeStruct((NC, 2, D), jnp.float32),
        ],
        scratch_shapes=[pltpu.VMEM((2, D), jnp.float32)],
    )(s_edges, stats, prm, w2)


def _node_body(x_ref, agg_ref, deg_ref, wx_ref, wfi_ref, wfo_ref, fcb_ref,
               w2_ref, b2_ref, xn_ref):
    x = x_ref[...]
    dn = (((1,), (1,)), ((), ()))
    dp = deg_ref[0, :, 0:1]
    dc = deg_ref[1, :, 0:1]
    fi = agg_ref[0] * jnp.where(dp > 0, 1.0 / dp, 0.0)
    fo = agg_ref[1] * jnp.where(dc > 0, 1.0 / dc, 0.0)
    h = lax.dot_general(x, wx_ref[...], dn, preferred_element_type=jnp.float32)
    h += lax.dot_general(fi, wfi_ref[...], dn,
                         preferred_element_type=jnp.float32)
    h += lax.dot_general(fo, wfo_ref[...], dn,
                         preferred_element_type=jnp.float32)
    h = jnp.maximum(h + fcb_ref[...], 0.0)
    xn = lax.dot_general(h, w2_ref[...], dn, preferred_element_type=jnp.float32)
    xn_ref[...] = x + xn + b2_ref[...]


def _node_update(x, agg, deg, wx, wfi, wfo, fcb, w2, b2):
    g = N // BN_NODE
    return pl.pallas_call(
        _node_body,
        grid=(g,),
        in_specs=[
            pl.BlockSpec((BN_NODE, D), lambda i: (i, 0)),
            pl.BlockSpec((NC, BN_NODE, D), lambda i: (0, i, 0)),
            pl.BlockSpec((NC, BN_NODE, DEGW), lambda i: (0, i, 0)),
            pl.BlockSpec((D2, D), lambda i: (0, 0)),
            pl.BlockSpec((D2, D), lambda i: (0, 0)),
            pl.BlockSpec((D2, D), lambda i: (0, 0)),
            pl.BlockSpec((1, D2), lambda i: (0, 0)),
            pl.BlockSpec((D, D2), lambda i: (0, 0)),
            pl.BlockSpec((1, D), lambda i: (0, 0)),
        ],
        out_specs=pl.BlockSpec((BN_NODE, D), lambda i: (i, 0)),
        out_shape=jax.ShapeDtypeStruct((N, D), jnp.float32),
    )(x, agg, deg, wx, wfi, wfo, fcb, w2, b2)


def _final_body(x_ref, w_ref, b_ref, o_ref):
    o_ref[...] = lax.dot_general(
        x_ref[...], w_ref[...], (((1,), (1,)), ((), ())),
        preferred_element_type=jnp.float32) + b_ref[...]


def _final_conv(x, w, b):
    g = N // BN_NODE
    return pl.pallas_call(
        _final_body,
        grid=(g,),
        in_specs=[
            pl.BlockSpec((BN_NODE, D), lambda i: (i, 0)),
            pl.BlockSpec((D2, D), lambda i: (0, 0)),
            pl.BlockSpec((1, D2), lambda i: (0, 0)),
        ],
        out_specs=pl.BlockSpec((BN_NODE, D2), lambda i: (i, 0)),
        out_shape=jax.ShapeDtypeStruct((N, D2), jnp.float32),
    )(x, w, b)


@jax.jit
def _run(nodes, edges, emb, parent_w1, parent_b1, parent_g1, parent_be1,
         parent_w2, parent_b2, parent_g2, parent_be2, child_w1, child_b1,
         child_g1, child_be1, child_w2, child_b2, child_g2, child_be2,
         fc_w, fc_b, fc2_w, fc2_b, conv_w, conv_b):
    nodes = nodes.astype(jnp.int32)
    edges = edges.astype(jnp.int32)
    e0 = edges[0]
    e1 = edges[1]

    wi = jnp.stack([parent_w1[:, :D], child_w1[:, :D]])
    wj = jnp.stack([parent_w1[:, D:], child_w1[:, D:]])
    prm = jnp.stack([
        jnp.stack([parent_g1, parent_be1, parent_g2, parent_be2]),
        jnp.stack([child_g1, child_be1, child_g2, child_be2]),
    ])
    w2 = jnp.stack([parent_w2, child_w2])
    wx = fc_w[:, :D]
    wfi = fc_w[:, D:D2]
    wfo = fc_w[:, D2:]
    fcb = fc_b.reshape(1, D2)
    fc2b = fc2_b.reshape(1, D)
    convb = conv_b.reshape(1, D2)

    x = _emb_gather(emb, nodes)
    deg = _deg_tables(e0, e1)

    for _ in range(2):
        ti, tj = _node_tables(x, wi, wj)
        s_edges, stats = _edge_gather(ti, tj, e0, e1)
        h2v, coef = _edge_mlp(s_edges, stats, prm, w2)
        agg = _edge_scatter(h2v, coef, e0, e1)
        x = _node_update(x, agg, deg, wx, wfi, wfo, fcb, fc2_w, fc2b)

    return _final_conv(x, conv_w, convb)


def kernel(nodes, edges, emb, parent_w1, parent_b1, parent_g1, parent_be1,
           parent_w2, parent_b2, parent_g2, parent_be2, child_w1, child_b1,
           child_g1, child_be1, child_w2, child_b2, child_g2, child_be2,
           fc_w, fc_b, fc2_w, fc2_b, conv_w, conv_b):
    return _run(nodes, edges, emb, parent_w1, parent_b1, parent_g1, parent_be1,
                parent_w2, parent_b2, parent_g2, parent_be2, child_w1,
                child_b1, child_g1, child_be1, child_w2, child_b2, child_g2,
                child_be2, fc_w, fc_b, fc2_w, fc2_b, conv_w, conv_b)
